# baseline (speedup 1.0000x reference)
import math

import jax
import jax.numpy as jnp
from jax import lax
from jax.experimental import pallas as pl
from jax.experimental.pallas import tpu as pltpu
from jax.experimental.pallas import tpu_sc as plsc

N = 50000
E = 1600000
DT = 5e-05
DASHPOT_DAMPING = 100.0
DRAG_DAMPING = 3.0

NC = 2
NS = 16
NW = NC * NS

NP = 50048
NP4 = NP * 4
RT4 = NP4 // NS
RWF = NP4 // 128

PW = E // NW
C1 = 2000
C2 = 400
N3 = 3 * N
BT4 = RT4 // 2

_MAGIC = 0x5F3759DF


def _rsqrt16(q):
    ib = lax.bitcast_convert_type(q, jnp.int32)
    y = lax.bitcast_convert_type(
        jnp.full((16,), _MAGIC, jnp.int32) - (ib >> 1), jnp.float32)
    for _ in range(3):
        y = y * (1.5 - 0.5 * q * y * y)
    return y


def _make_edge_kernel(with_v, C):
    C3 = 3 * C
    C6 = 6 * C
    NCH = PW // C

    mesh = plsc.VectorSubcoreMesh(
        core_axis_name="c", subcore_axis_name="s", num_cores=NC, num_subcores=NS
    )

    if with_v:
        out_type = (jax.ShapeDtypeStruct((NC * NP4,), jnp.float32),
                    jax.ShapeDtypeStruct((3 * E,), jnp.float32))
    else:
        out_type = (jax.ShapeDtypeStruct((NC * NP4,), jnp.float32),
                    jax.ShapeDtypeStruct((NP4,), jnp.float32))

    f32 = jnp.float32
    i32 = jnp.int32

    def one_set():
        s = [
            pltpu.VMEM((C,), i32),
            pltpu.VMEM((C,), i32),
            pltpu.VMEM((C,), f32),
            pltpu.VMEM((C,), f32),
            pltpu.VMEM((C6,), i32),
            pltpu.VMEM((C6,), f32),
            pltpu.VMEM((C6,), f32),
        ]
        if with_v:
            s += [
                pltpu.VMEM((C6,), f32),
                pltpu.VMEM((C3,), f32),
            ]
        s += [pltpu.SemaphoreType.DMA] * 3
        return s

    NSET = len(one_set())
    scratch = one_set() + one_set() + [
        pltpu.VMEM((BT4,), f32),
        pltpu.VMEM_SHARED((NP4,), f32),
        pltpu.VMEM_SHARED((NP4,), f32),
    ]
    if with_v:
        scratch.append(pltpu.VMEM_SHARED((NP4,), f32))
    else:
        scratch.append(pltpu.VMEM((BT4,), i32))
        scratch.append(pltpu.SemaphoreType.DMA)

    class Set:
        def __init__(self, refs):
            (self.idxf1, self.idxf2, self.rest_v, self.y_v,
             self.ia, self.xp, self.src) = refs[:7]
            if with_v:
                self.vp, self.sf_s = refs[7:9]
            self.lsem, self.gsem, self.ssem = refs[NSET - 3:NSET]

    n_in = 7 if with_v else 6
    n_out = 2

    def body(*refs):
        if with_v:
            x4, v4, i1f, i2f, rest, yy, zeros = refs[:n_in]
            f_out, sf_out = refs[n_in:n_in + n_out]
        else:
            x4, i1f, i2f, rest, yy, zeros = refs[:n_in]
            f_out, x4_out = refs[n_in:n_in + n_out]
        sc = refs[n_in + n_out:]
        A = Set(sc[:NSET])
        B = Set(sc[NSET:2 * NSET])
        bounce = sc[2 * NSET]
        shared_x = sc[2 * NSET + 1]
        shared_f = sc[2 * NSET + 2]
        if with_v:
            shared_v = sc[2 * NSET + 3]
        else:
            shared_v = None
            sidx, stsem = sc[2 * NSET + 3:2 * NSET + 5]

        c = lax.axis_index("c")
        s = lax.axis_index("s")
        wid = s * NC + c

        tsl = pl.ds(s * RT4, RT4)
        lanes16 = lax.iota(jnp.int32, 16)
        for j in range(2):
            psl = pl.ds(s * RT4 + j * BT4, BT4)
            if with_v:
                pltpu.sync_copy(x4.at[psl], bounce)
                pltpu.sync_copy(bounce, shared_x.at[psl])
                pltpu.sync_copy(v4.at[psl], bounce)
                pltpu.sync_copy(bounce, shared_v.at[psl])
            else:
                @pl.loop(0, BT4 // 16)
                def _(it):
                    w = s * RT4 + j * BT4 + it * 16 + lanes16
                    v16 = w >> 2
                    cmp = w & 3
                    sidx[pl.ds(it * 16, 16)] = jnp.where(
                        cmp == 3, jnp.full((16,), N3, jnp.int32),
                        v16 * 3 + cmp)
                pltpu.async_copy(x4.at[sidx], bounce, stsem).wait()
                pltpu.sync_copy(bounce, shared_x.at[psl])
                pltpu.sync_copy(bounce, x4_out.at[psl])
            pltpu.sync_copy(zeros, bounce)
            pltpu.sync_copy(bounce, shared_f.at[psl])

        plsc.subcore_barrier()

        def fire_loads(ci, S):
            row0 = wid * PW + ci * C
            pltpu.async_copy(i1f.at[pl.ds(row0, C)], S.idxf1, S.lsem)
            pltpu.async_copy(i2f.at[pl.ds(row0, C)], S.idxf2, S.lsem)
            pltpu.async_copy(rest.at[pl.ds(row0, C)], S.rest_v, S.lsem)
            pltpu.async_copy(yy.at[pl.ds(row0, C)], S.y_v, S.lsem)

        def sync_loads(ci, S):
            row0 = wid * PW + ci * C
            pltpu.sync_copy(i1f.at[pl.ds(row0, C)], S.idxf1)
            pltpu.sync_copy(i2f.at[pl.ds(row0, C)], S.idxf2)
            pltpu.sync_copy(rest.at[pl.ds(row0, C)], S.rest_v)
            pltpu.sync_copy(yy.at[pl.ds(row0, C)], S.y_v)

        def idxgen(S):
            @pl.loop(0, C // 16)
            def _(it):
                sl = pl.ds(it * 16, 16)
                b1 = S.idxf1[sl] << 2
                b2 = S.idxf2[sl] << 2
                S.ia[sl] = b1
                S.ia[pl.ds(C + it * 16, 16)] = b1 + 1
                S.ia[pl.ds(2 * C + it * 16, 16)] = b1 + 2
                S.ia[pl.ds(3 * C + it * 16, 16)] = b2
                S.ia[pl.ds(4 * C + it * 16, 16)] = b2 + 1
                S.ia[pl.ds(5 * C + it * 16, 16)] = b2 + 2

        def fire_gathers(S):
            pltpu.async_copy(shared_x.at[S.ia], S.xp, S.gsem)
            if with_v:
                pltpu.async_copy(shared_v.at[S.ia], S.vp, S.gsem)

        def compute(S):
            @pl.loop(0, C // 16)
            def _(it):
                e0 = it * 16
                sl = pl.ds(e0, 16)
                sly = pl.ds(C + e0, 16)
                slz = pl.ds(2 * C + e0, 16)
                sl2x = pl.ds(3 * C + e0, 16)
                sl2y = pl.ds(4 * C + e0, 16)
                sl2z = pl.ds(5 * C + e0, 16)
                dx = S.xp[sl2x] - S.xp[sl]
                dy = S.xp[sl2y] - S.xp[sly]
                dz = S.xp[sl2z] - S.xp[slz]
                q = dx * dx + dy * dy + dz * dz
                rinv = _rsqrt16(q)
                nrm = q * rinv
                k16 = jnp.exp(S.y_v[sl])
                coef_s = k16 * (nrm / S.rest_v[sl] - 1.0)
                if with_v:
                    vrel = ((S.vp[sl2x] - S.vp[sl]) * dx
                            + (S.vp[sl2y] - S.vp[sly]) * dy
                            + (S.vp[sl2z] - S.vp[slz]) * dz) * rinv
                    coef = coef_s + DASHPOT_DAMPING * vrel
                else:
                    coef = coef_s
                cc = coef * rinv
                fx = cc * dx
                fy = cc * dy
                fz = cc * dz
                S.src[sl] = fx
                S.src[sly] = fy
                S.src[slz] = fz
                S.src[sl2x] = -fx
                S.src[sl2y] = -fy
                S.src[sl2z] = -fz
                if with_v:
                    sv = coef_s * rinv
                    S.sf_s[sl] = sv * dx
                    S.sf_s[sly] = sv * dy
                    S.sf_s[slz] = sv * dz

        def fire_scatters(ci, S, async_sc=False):
            pltpu.sync_copy(S.src, shared_f.at[S.ia], add=True)
            if with_v:
                g = wid * NCH + ci
                pltpu.async_copy(S.sf_s, sf_out.at[pl.ds(g * C3, C3)], S.ssem)
            return []

        def drain_gathers(S):
            pltpu.make_async_copy(shared_x.at[S.ia], S.xp, S.gsem).wait()
            if with_v:
                pltpu.make_async_copy(shared_v.at[S.ia], S.vp, S.gsem).wait()

        def drain_scatters(S, ci):
            if with_v:
                g = wid * NCH + ci
                pltpu.make_async_copy(
                    S.sf_s, sf_out.at[pl.ds(g * C3, C3)], S.ssem).wait()

        def drain_loads(S, ci):
            row0 = wid * PW + ci * C
            pltpu.make_async_copy(i1f.at[pl.ds(row0, C)], S.idxf1, S.lsem).wait()
            pltpu.make_async_copy(i2f.at[pl.ds(row0, C)], S.idxf2, S.lsem).wait()
            pltpu.make_async_copy(rest.at[pl.ds(row0, C)], S.rest_v, S.lsem).wait()
            pltpu.make_async_copy(yy.at[pl.ds(row0, C)], S.y_v, S.lsem).wait()

        def section(ci, S, Snext, fire_next, has_prev, async_sc=False):
            if fire_next:
                fire_loads(ci + 1, Snext)
            drain_gathers(S)
            compute(S)
            h = fire_scatters(ci, S, async_sc)
            if has_prev:
                drain_scatters(Snext, ci - 1)
            if fire_next:
                drain_loads(Snext, ci + 1)
                idxgen(Snext)
                fire_gathers(Snext)
            return h

        sync_loads(0, A)
        idxgen(A)
        fire_gathers(A)
        section(0, A, B, True, False)

        K = (NCH - 3) // 2

        @pl.loop(0, K)
        def _(k):
            section(2 * k + 1, B, A, True, True)
            section(2 * k + 2, A, B, True, True)

        section(NCH - 2, B, A, True, True)
        section(NCH - 1, A, B, False, True)
        drain_scatters(A, NCH - 1)

        plsc.subcore_barrier()
        for j in range(2):
            pltpu.sync_copy(shared_f.at[pl.ds(s * RT4 + j * BT4, BT4)], bounce)
            pltpu.sync_copy(
                bounce, f_out.at[pl.ds(c * NP4 + s * RT4 + j * BT4, BT4)])

    kern = pl.kernel(
        body, out_type=out_type, mesh=mesh, scratch_types=scratch,
        name="edge_pass_v" if with_v else "edge_pass",
    )
    return kern


_edge_nov = _make_edge_kernel(False, C1)
_edge_v = _make_edge_kernel(True, C2)


def _integrate_body(x_ref, v_ref, f0_ref, f1_ref, m_ref, xo_ref, vo_ref):
    damp = math.exp(-DT * DRAG_DAMPING)
    lane = lax.broadcasted_iota(jnp.int32, (RWF, 128), 1)
    m2 = (lane % 4) == 2
    g = jnp.where(m2, jnp.float32(-9.8), jnp.float32(0.0))
    acc = (f0_ref[...] + f1_ref[...]) / m_ref[...] + g
    vn = (v_ref[...] + DT * acc) * damp
    xn = x_ref[...] + DT * vn
    xc = jnp.where(m2, jnp.maximum(xn, 0.0), xn)
    vz = jnp.where(m2 & (xc == 0.0), jnp.float32(0.0), vn)
    xo_ref[...] = xc
    vo_ref[...] = vz


_integrate = pl.pallas_call(
    _integrate_body,
    out_shape=(
        jax.ShapeDtypeStruct((RWF, 128), jnp.float32),
        jax.ShapeDtypeStruct((RWF, 128), jnp.float32),
    ),
)


@jax.jit
def kernel(init_vertices, init_springs, init_rest_lengths, init_masses, spring_Y):
    f32 = jnp.float32
    i32 = jnp.int32

    i1f = init_springs[:, 0].astype(i32)
    i2f = init_springs[:, 1].astype(i32)
    restp = init_rest_lengths.astype(f32)
    yp = spring_Y.astype(f32)

    x3z = jnp.concatenate([init_vertices.astype(f32).reshape(N3),
                           jnp.zeros((1,), f32)])
    masses_p = jnp.concatenate([init_masses.astype(f32), jnp.ones((NP - N,), f32)])
    m4f = jnp.broadcast_to(masses_p[:, None], (NP, 4)).reshape(RWF, 128)
    zeros_tile = jnp.zeros((BT4,), f32)

    fp1, x4f = _edge_nov(x3z, i1f, i2f, restp, yp, zeros_tile)
    f0 = fp1[:NP4].reshape(RWF, 128)
    f1 = fp1[NP4:].reshape(RWF, 128)
    xf, vf = _integrate(x4f.reshape(RWF, 128), jnp.zeros((RWF, 128), f32), f0, f1,
                        m4f)

    fp2, sf3 = _edge_v(xf.reshape(NP4), vf.reshape(NP4), i1f, i2f, restp, yp,
                       zeros_tile)
    f0b = fp2[:NP4].reshape(RWF, 128)
    f1b = fp2[NP4:].reshape(RWF, 128)
    xf2, _ = _integrate(xf, vf, f0b, f1b, m4f)

    x_out = xf2.reshape(NP, 4)[:N, :3]
    sf_out = sf3.reshape(NW * (PW // C2), 3, C2).transpose(0, 2, 1).reshape(E, 3)
    return (x_out, init_springs, init_rest_lengths, sf_out)


# speedup vs baseline: 1.3021x; 1.3021x over previous
import math

import jax
import jax.numpy as jnp
from jax import lax
from jax.experimental import pallas as pl
from jax.experimental.pallas import tpu as pltpu
from jax.experimental.pallas import tpu_sc as plsc

N = 50000
E = 1600000
DT = 5e-05
DASHPOT_DAMPING = 100.0
DRAG_DAMPING = 3.0

NC = 2
NS = 16
NW = NC * NS

NP = 50048
NP4 = NP * 4
RT4 = NP4 // NS
RWF = NP4 // 128

PW = E // NW
C1 = 2000
C2 = 400

_MAGIC = 0x5F3759DF


def _rsqrt16(q):
    ib = lax.bitcast_convert_type(q, jnp.int32)
    y = lax.bitcast_convert_type(
        jnp.full((16,), _MAGIC, jnp.int32) - (ib >> 1), jnp.float32)
    for _ in range(3):
        y = y * (1.5 - 0.5 * q * y * y)
    return y


def _make_edge_kernel(with_v, C):
    C3 = 3 * C
    C6 = 6 * C
    NCH = PW // C

    mesh = plsc.VectorSubcoreMesh(
        core_axis_name="c", subcore_axis_name="s", num_cores=NC, num_subcores=NS
    )

    if with_v:
        out_type = (jax.ShapeDtypeStruct((NC * NP4,), jnp.float32),
                    jax.ShapeDtypeStruct((3 * E,), jnp.float32))
    else:
        out_type = jax.ShapeDtypeStruct((NC * NP4,), jnp.float32)

    f32 = jnp.float32
    i32 = jnp.int32

    def one_set():
        s = [
            pltpu.VMEM((C,), i32),
            pltpu.VMEM((C,), i32),
            pltpu.VMEM((C,), f32),
            pltpu.VMEM((C,), f32),
            pltpu.VMEM((C6,), i32),
            pltpu.VMEM((C6,), f32),
            pltpu.VMEM((C6,), f32),
        ]
        if with_v:
            s += [
                pltpu.VMEM((C6,), f32),
                pltpu.VMEM((C3,), f32),
            ]
        s += [pltpu.SemaphoreType.DMA] * 3
        return s

    NSET = len(one_set())
    scratch = one_set() + one_set() + [
        pltpu.VMEM((RT4,), f32),
        pltpu.VMEM_SHARED((NP4,), f32),
        pltpu.VMEM_SHARED((NP4,), f32),
    ]
    if with_v:
        scratch.append(pltpu.VMEM_SHARED((NP4,), f32))

    class Set:
        def __init__(self, refs):
            (self.idxf1, self.idxf2, self.rest_v, self.y_v,
             self.ia, self.xp, self.src) = refs[:7]
            if with_v:
                self.vp, self.sf_s = refs[7:9]
            self.lsem, self.gsem, self.ssem = refs[NSET - 3:NSET]

    n_in = 7 if with_v else 6
    n_out = 2 if with_v else 1

    def body(*refs):
        if with_v:
            x4, v4, i1f, i2f, rest, yy, zeros = refs[:n_in]
            f_out, sf_out = refs[n_in:n_in + n_out]
        else:
            x4, i1f, i2f, rest, yy, zeros = refs[:n_in]
            (f_out,) = refs[n_in:n_in + n_out]
        sc = refs[n_in + n_out:]
        A = Set(sc[:NSET])
        B = Set(sc[NSET:2 * NSET])
        bounce = sc[2 * NSET]
        shared_x = sc[2 * NSET + 1]
        shared_f = sc[2 * NSET + 2]
        shared_v = sc[2 * NSET + 3] if with_v else None

        c = lax.axis_index("c")
        s = lax.axis_index("s")
        wid = s * NC + c

        tsl = pl.ds(s * RT4, RT4)
        pltpu.sync_copy(x4.at[tsl], bounce)
        pltpu.sync_copy(bounce, shared_x.at[tsl])
        if with_v:
            pltpu.sync_copy(v4.at[tsl], bounce)
            pltpu.sync_copy(bounce, shared_v.at[tsl])
        pltpu.sync_copy(zeros, bounce)
        pltpu.sync_copy(bounce, shared_f.at[tsl])

        plsc.subcore_barrier()

        def fire_loads(ci, S):
            row0 = wid * PW + ci * C
            pltpu.async_copy(i1f.at[pl.ds(row0, C)], S.idxf1, S.lsem)
            pltpu.async_copy(i2f.at[pl.ds(row0, C)], S.idxf2, S.lsem)
            pltpu.async_copy(rest.at[pl.ds(row0, C)], S.rest_v, S.lsem)
            pltpu.async_copy(yy.at[pl.ds(row0, C)], S.y_v, S.lsem)

        def sync_loads(ci, S):
            row0 = wid * PW + ci * C
            pltpu.sync_copy(i1f.at[pl.ds(row0, C)], S.idxf1)
            pltpu.sync_copy(i2f.at[pl.ds(row0, C)], S.idxf2)
            pltpu.sync_copy(rest.at[pl.ds(row0, C)], S.rest_v)
            pltpu.sync_copy(yy.at[pl.ds(row0, C)], S.y_v)

        def idxgen(S):
            @pl.loop(0, C // 16)
            def _(it):
                sl = pl.ds(it * 16, 16)
                b1 = S.idxf1[sl] << 2
                b2 = S.idxf2[sl] << 2
                S.ia[sl] = b1
                S.ia[pl.ds(C + it * 16, 16)] = b1 + 1
                S.ia[pl.ds(2 * C + it * 16, 16)] = b1 + 2
                S.ia[pl.ds(3 * C + it * 16, 16)] = b2
                S.ia[pl.ds(4 * C + it * 16, 16)] = b2 + 1
                S.ia[pl.ds(5 * C + it * 16, 16)] = b2 + 2

        def fire_gathers(S):
            pltpu.async_copy(shared_x.at[S.ia], S.xp, S.gsem)
            if with_v:
                pltpu.async_copy(shared_v.at[S.ia], S.vp, S.gsem)

        def compute(S):
            @pl.loop(0, C // 16)
            def _(it):
                e0 = it * 16
                sl = pl.ds(e0, 16)
                sly = pl.ds(C + e0, 16)
                slz = pl.ds(2 * C + e0, 16)
                sl2x = pl.ds(3 * C + e0, 16)
                sl2y = pl.ds(4 * C + e0, 16)
                sl2z = pl.ds(5 * C + e0, 16)
                dx = S.xp[sl2x] - S.xp[sl]
                dy = S.xp[sl2y] - S.xp[sly]
                dz = S.xp[sl2z] - S.xp[slz]
                q = dx * dx + dy * dy + dz * dz
                rinv = _rsqrt16(q)
                nrm = q * rinv
                k16 = jnp.exp(S.y_v[sl])
                coef_s = k16 * (nrm / S.rest_v[sl] - 1.0)
                if with_v:
                    vrel = ((S.vp[sl2x] - S.vp[sl]) * dx
                            + (S.vp[sl2y] - S.vp[sly]) * dy
                            + (S.vp[sl2z] - S.vp[slz]) * dz) * rinv
                    coef = coef_s + DASHPOT_DAMPING * vrel
                else:
                    coef = coef_s
                cc = coef * rinv
                fx = cc * dx
                fy = cc * dy
                fz = cc * dz
                S.src[sl] = fx
                S.src[sly] = fy
                S.src[slz] = fz
                S.src[sl2x] = -fx
                S.src[sl2y] = -fy
                S.src[sl2z] = -fz
                if with_v:
                    sv = coef_s * rinv
                    S.sf_s[sl] = sv * dx
                    S.sf_s[sly] = sv * dy
                    S.sf_s[slz] = sv * dz

        def fire_scatters(ci, S, async_sc=False):
            pltpu.sync_copy(S.src, shared_f.at[S.ia], add=True)
            if with_v:
                g = wid * NCH + ci
                pltpu.async_copy(S.sf_s, sf_out.at[pl.ds(g * C3, C3)], S.ssem)
            return []

        def drain_gathers(S):
            pltpu.make_async_copy(shared_x.at[S.ia], S.xp, S.gsem).wait()
            if with_v:
                pltpu.make_async_copy(shared_v.at[S.ia], S.vp, S.gsem).wait()

        def drain_scatters(S, ci):
            if with_v:
                g = wid * NCH + ci
                pltpu.make_async_copy(
                    S.sf_s, sf_out.at[pl.ds(g * C3, C3)], S.ssem).wait()

        def drain_loads(S, ci):
            row0 = wid * PW + ci * C
            pltpu.make_async_copy(i1f.at[pl.ds(row0, C)], S.idxf1, S.lsem).wait()
            pltpu.make_async_copy(i2f.at[pl.ds(row0, C)], S.idxf2, S.lsem).wait()
            pltpu.make_async_copy(rest.at[pl.ds(row0, C)], S.rest_v, S.lsem).wait()
            pltpu.make_async_copy(yy.at[pl.ds(row0, C)], S.y_v, S.lsem).wait()

        def section(ci, S, Snext, fire_next, has_prev, async_sc=False):
            if fire_next:
                fire_loads(ci + 1, Snext)
            drain_gathers(S)
            compute(S)
            h = fire_scatters(ci, S, async_sc)
            if has_prev:
                drain_scatters(Snext, ci - 1)
            if fire_next:
                drain_loads(Snext, ci + 1)
                idxgen(Snext)
                fire_gathers(Snext)
            return h

        sync_loads(0, A)
        idxgen(A)
        fire_gathers(A)
        section(0, A, B, True, False)

        K = (NCH - 3) // 2

        @pl.loop(0, K)
        def _(k):
            section(2 * k + 1, B, A, True, True)
            section(2 * k + 2, A, B, True, True)

        section(NCH - 2, B, A, True, True)
        section(NCH - 1, A, B, False, True)
        drain_scatters(A, NCH - 1)

        plsc.subcore_barrier()
        pltpu.sync_copy(shared_f.at[tsl], bounce)
        pltpu.sync_copy(bounce, f_out.at[pl.ds(c * NP4 + s * RT4, RT4)])

    kern = pl.kernel(
        body, out_type=out_type, mesh=mesh, scratch_types=scratch,
        name="edge_pass_v" if with_v else "edge_pass",
    )
    return kern


_edge_nov = _make_edge_kernel(False, C1)
_edge_v = _make_edge_kernel(True, C2)


def _integrate_body(x_ref, v_ref, f0_ref, f1_ref, m_ref, xo_ref, vo_ref):
    damp = math.exp(-DT * DRAG_DAMPING)
    lane = lax.broadcasted_iota(jnp.int32, (RWF, 128), 1)
    m2 = (lane % 4) == 2
    g = jnp.where(m2, jnp.float32(-9.8), jnp.float32(0.0))
    acc = (f0_ref[...] + f1_ref[...]) / m_ref[...] + g
    vn = (v_ref[...] + DT * acc) * damp
    xn = x_ref[...] + DT * vn
    xc = jnp.where(m2, jnp.maximum(xn, 0.0), xn)
    vz = jnp.where(m2 & (xc == 0.0), jnp.float32(0.0), vn)
    xo_ref[...] = xc
    vo_ref[...] = vz


_integrate = pl.pallas_call(
    _integrate_body,
    out_shape=(
        jax.ShapeDtypeStruct((RWF, 128), jnp.float32),
        jax.ShapeDtypeStruct((RWF, 128), jnp.float32),
    ),
)


@jax.jit
def kernel(init_vertices, init_springs, init_rest_lengths, init_masses, spring_Y):
    f32 = jnp.float32
    i32 = jnp.int32

    i1f = init_springs[:, 0].astype(i32)
    i2f = init_springs[:, 1].astype(i32)
    restp = init_rest_lengths.astype(f32)
    yp = spring_Y.astype(f32)

    x4 = jnp.zeros((NP, 4), f32).at[:N, :3].set(init_vertices.astype(f32))
    x4f = x4.reshape(NP4)
    masses_p = jnp.concatenate([init_masses.astype(f32), jnp.ones((NP - N,), f32)])
    m4f = jnp.broadcast_to(masses_p[:, None], (NP, 4)).reshape(RWF, 128)
    zeros_tile = jnp.zeros((RT4,), f32)

    fp1 = _edge_nov(x4f, i1f, i2f, restp, yp, zeros_tile)
    f0 = fp1[:NP4].reshape(RWF, 128)
    f1 = fp1[NP4:].reshape(RWF, 128)
    xf, vf = _integrate(x4f.reshape(RWF, 128), jnp.zeros((RWF, 128), f32), f0, f1,
                        m4f)

    fp2, sf3 = _edge_v(xf.reshape(NP4), vf.reshape(NP4), i1f, i2f, restp, yp,
                       zeros_tile)
    f0b = fp2[:NP4].reshape(RWF, 128)
    f1b = fp2[NP4:].reshape(RWF, 128)
    xf2, _ = _integrate(xf, vf, f0b, f1b, m4f)

    x_out = xf2.reshape(NP, 4)[:N, :3]
    sf_out = sf3.reshape(NW * (PW // C2), 3, C2).transpose(0, 2, 1).reshape(E, 3)
    return (x_out, init_springs, init_rest_lengths, sf_out)
